# Initial kernel scaffold; baseline (speedup 1.0000x reference)
import functools

import jax
import jax.numpy as jnp
from jax import lax
from jax.experimental import pallas as pl
from jax.experimental.pallas import tpu as pltpu
from jax.experimental.pallas import tpu_sc as plsc

N_NODES = 10000
N_EDGES = 160000
M = 16
MID = 32
OUT3 = 256

NC = 2
NS = 16
NW = NC * NS
CHUNK = 128
PW_PAD = 5120
NCHUNK = PW_PAD // CHUNK
E_PAD = NW * PW_PAD
N_PAD = 10240
ZROWS = N_PAD // NS

EB = 2048
FIRE = 8


def _sc_gather(row_idx, h0f):
    mesh = plsc.VectorSubcoreMesh(core_axis_name="c", subcore_axis_name="s")

    @functools.partial(
        pl.kernel,
        out_type=jax.ShapeDtypeStruct((E_PAD, M), jnp.float32),
        mesh=mesh,
        scratch_types=[
            pltpu.VMEM((NCHUNK, CHUNK), jnp.int32),
            pltpu.VMEM((PW_PAD, M), jnp.float32),
            pltpu.SemaphoreType.DMA,
        ],
    )
    def gather_kernel(row_hbm, h0_hbm, src_hbm, idx_v, rows_v, sem):
        cid = lax.axis_index("c")
        sid = lax.axis_index("s")
        wid = sid * NC + cid
        pltpu.sync_copy(row_hbm.at[wid], idx_v)

        def group(g, carry):
            base = g * FIRE
            descs = []
            for k in range(FIRE):
                j = base + k
                descs.append(
                    pltpu.async_copy(
                        h0_hbm.at[idx_v.at[j]],
                        rows_v.at[pl.ds(j * CHUNK, CHUNK)],
                        sem,
                    )
                )
            for d in descs:
                d.wait()
            return carry

        lax.fori_loop(0, NCHUNK // FIRE, group, 0)
        pltpu.sync_copy(rows_v, src_hbm.at[pl.ds(wid * PW_PAD, PW_PAD)])

    return gather_kernel(row_idx, h0f)


def _sc_scatter(msg, col_idx):
    mesh = plsc.VectorSubcoreMesh(core_axis_name="c", subcore_axis_name="s")

    @functools.partial(
        pl.kernel,
        out_type=(
            jax.ShapeDtypeStruct((NC * N_PAD, M), jnp.float32),
            jax.ShapeDtypeStruct((NC * N_PAD, M), jnp.float32),
        ),
        mesh=mesh,
        scratch_types=[
            pltpu.VMEM((NCHUNK, CHUNK), jnp.int32),
            pltpu.VMEM((PW_PAD, M), jnp.float32),
            pltpu.VMEM((CHUNK, M), jnp.float32),
            pltpu.VMEM((ZROWS, M), jnp.float32),
            pltpu.VMEM_SHARED((N_PAD, M), jnp.float32),
            pltpu.VMEM_SHARED((N_PAD, M), jnp.float32),
        ],
    )
    def scatter_kernel(msg_hbm, col_hbm, seg_hbm, cnt_hbm,
                       idx_v, msg_v, ones_v, zz_v, seg_sp, cnt_sp):
        cid = lax.axis_index("c")
        sid = lax.axis_index("s")
        wid = sid * NC + cid

        zrow = jnp.zeros((M,), jnp.float32)

        def zbody(i, c):
            zz_v[i, :] = zrow
            return c

        lax.fori_loop(0, ZROWS, zbody, 0)

        orow = jnp.ones((M,), jnp.float32)

        def obody(i, c):
            ones_v[i, :] = orow
            return c

        lax.fori_loop(0, CHUNK, obody, 0)

        rbase = sid * ZROWS
        pltpu.sync_copy(zz_v, seg_sp.at[pl.ds(rbase, ZROWS)])
        pltpu.sync_copy(zz_v, cnt_sp.at[pl.ds(rbase, ZROWS)])
        pltpu.sync_copy(col_hbm.at[wid], idx_v)
        pltpu.sync_copy(msg_hbm.at[pl.ds(wid * PW_PAD, PW_PAD)], msg_v)
        plsc.subcore_barrier()

        def sbody(j, c):
            pltpu.sync_copy(msg_v.at[pl.ds(j * CHUNK, CHUNK)],
                            seg_sp.at[idx_v.at[j]], add=True)
            pltpu.sync_copy(ones_v, cnt_sp.at[idx_v.at[j]], add=True)
            return c

        lax.fori_loop(0, NCHUNK, sbody, 0)
        plsc.subcore_barrier()

        pltpu.sync_copy(seg_sp.at[pl.ds(rbase, ZROWS)], zz_v)
        pltpu.sync_copy(zz_v, seg_hbm.at[pl.ds(cid * N_PAD + rbase, ZROWS)])
        pltpu.sync_copy(cnt_sp.at[pl.ds(rbase, ZROWS)], zz_v)
        pltpu.sync_copy(zz_v, cnt_hbm.at[pl.ds(cid * N_PAD + rbase, ZROWS)])

    return scatter_kernel(msg, col_idx)


def _ln_relu(x, g, b):
    mu = jnp.mean(x, axis=-1, keepdims=True)
    var = jnp.mean((x - mu) * (x - mu), axis=-1, keepdims=True)
    y = (x - mu) * lax.rsqrt(var + 1e-5) * g + b
    return jnp.maximum(y, 0.0)


def _tc_messages(efp, rp, basp, src, W1a, W1b, b1, g1, be1,
                 W2, b2, g2, be2, W3, b3, T, S):
    def body(ef_ref, r_ref, bas_ref, src_ref, w1a, w1b, b1r, g1r, be1r,
             w2, b2r, g2r, be2r, w3, b3r, t, s, out_ref):
        x = jnp.dot(ef_ref[...], w1a[...], preferred_element_type=jnp.float32)
        x = x + r_ref[...] * w1b[...] + b1r[...]
        x = _ln_relu(x, g1r[...], be1r[...])
        x = jnp.dot(x, w2[...], preferred_element_type=jnp.float32) + b2r[...]
        x = _ln_relu(x, g2r[...], be2r[...])
        y = jnp.dot(x, w3[...], preferred_element_type=jnp.float32) + b3r[...]
        srep = jnp.dot(src_ref[...], t[...], preferred_element_type=jnp.float32)
        m = jnp.dot(y * srep, s[...], preferred_element_type=jnp.float32)
        out_ref[...] = m * bas_ref[...]

    eb16 = pl.BlockSpec((EB, M), lambda i: (i, 0))
    eb1 = pl.BlockSpec((EB, 1), lambda i: (i, 0))

    def wspec(a):
        return pl.BlockSpec(a.shape, lambda i: (0,) * a.ndim)

    weights = (W1a, W1b, b1, g1, be1, W2, b2, g2, be2, W3, b3, T, S)
    return pl.pallas_call(
        body,
        grid=(E_PAD // EB,),
        in_specs=[eb16, eb1, eb1, eb16] + [wspec(w) for w in weights],
        out_specs=eb16,
        out_shape=jax.ShapeDtypeStruct((E_PAD, M), jnp.float32),
    )(efp, rp, basp, src, *weights)


def _tc_finalize(seg2, cnt2, h0p, wst):
    NB = 4
    RB = N_PAD // NB

    def body(seg_ref, cnt_ref, h0_ref, w_ref, out_ref):
        sg = jnp.sum(seg_ref[...], axis=0)
        ct = jnp.sum(cnt_ref[...], axis=0)
        sf = jnp.dot(h0_ref[...], w_ref[...], preferred_element_type=jnp.float32)
        out_ref[...] = sg / jnp.maximum(ct, 1.0) + jnp.where(ct > 0.0, sf, 0.0)

    seg_spec = pl.BlockSpec((NC, RB, M), lambda i: (0, i, 0))
    row_spec = pl.BlockSpec((RB, M), lambda i: (i, 0))
    w_spec = pl.BlockSpec((M, M), lambda i: (0, 0))
    return pl.pallas_call(
        body,
        grid=(NB,),
        in_specs=[seg_spec, seg_spec, row_spec, w_spec],
        out_specs=row_spec,
        out_shape=jax.ShapeDtypeStruct((N_PAD, M), jnp.float32),
    )(seg2, cnt2, h0p, wst)


def kernel(h0, r, basis_00, edge_index, edge_feat,
           W1, b1, g1, be1, W2, b2, g2, be2, W3, b3, W_self):
    h0f = h0.reshape(N_NODES, M)
    row = edge_index[0]
    col = edge_index[1]
    pad = E_PAD - N_EDGES
    rowp = jnp.concatenate([row, jnp.zeros((pad,), jnp.int32)])
    rowp = rowp.reshape(NW, NCHUNK, CHUNK)
    colp = jnp.concatenate([col, jnp.full((pad,), N_NODES, jnp.int32)])
    colp = colp.reshape(NW, NCHUNK, CHUNK)
    efp = jnp.concatenate([edge_feat, jnp.zeros((pad, M), jnp.float32)])
    rp = jnp.concatenate([r, jnp.zeros((pad, 1), jnp.float32)])
    basp = jnp.concatenate(
        [basis_00.reshape(N_EDGES, 1), jnp.zeros((pad, 1), jnp.float32)])

    src = _sc_gather(rowp, h0f)

    k = jnp.arange(OUT3)
    T = (k[None, :] % M == jnp.arange(M)[:, None]).astype(jnp.float32)
    S = (k[:, None] // M == jnp.arange(M)[None, :]).astype(jnp.float32)
    msg = _tc_messages(
        efp, rp, basp, src,
        W1[:M], W1[M:], b1.reshape(1, MID), g1.reshape(1, MID),
        be1.reshape(1, MID), W2, b2.reshape(1, MID), g2.reshape(1, MID),
        be2.reshape(1, MID), W3, b3.reshape(1, OUT3), T, S)

    seg2, cnt2 = _sc_scatter(msg, colp)

    h0p = jnp.concatenate(
        [h0f, jnp.zeros((N_PAD - N_NODES, M), jnp.float32)])
    outp = _tc_finalize(seg2.reshape(NC, N_PAD, M),
                        cnt2.reshape(NC, N_PAD, M), h0p, W_self[0].T)
    return outp[:N_NODES].reshape(N_NODES, M, 1)


# speedup vs baseline: 2.6399x; 2.6399x over previous
import functools

import jax
import jax.numpy as jnp
from jax import lax
from jax.experimental import pallas as pl
from jax.experimental.pallas import tpu as pltpu
from jax.experimental.pallas import tpu_sc as plsc

N_NODES = 10000
N_EDGES = 160000
M = 16
MID = 32
OUT3 = 256

NC = 2
NS = 16
NW = NC * NS
CHUNK = 128
PW_PAD = 5120
NCHUNK = PW_PAD // CHUNK
E_PAD = NW * PW_PAD
N_PAD = 10240
ZROWS = N_PAD // NS

EB = 2048
FIRE = 8


def _sc_gather(row_idx, h0f):
    mesh = plsc.VectorSubcoreMesh(core_axis_name="c", subcore_axis_name="s")

    @functools.partial(
        pl.kernel,
        out_type=jax.ShapeDtypeStruct((E_PAD, M), jnp.float32),
        mesh=mesh,
        compiler_params=pltpu.CompilerParams(use_tc_tiling_on_sc=False),
        scratch_types=[
            pltpu.VMEM((NCHUNK, CHUNK), jnp.int32),
            pltpu.VMEM((PW_PAD, M), jnp.float32),
            pltpu.SemaphoreType.DMA,
        ],
    )
    def gather_kernel(row_hbm, h0_hbm, src_hbm, idx_v, rows_v, sem):
        cid = lax.axis_index("c")
        sid = lax.axis_index("s")
        wid = sid * NC + cid
        pltpu.sync_copy(row_hbm.at[wid], idx_v)

        def group(g, carry):
            base = g * FIRE
            descs = []
            for k in range(FIRE):
                j = base + k
                descs.append(
                    pltpu.async_copy(
                        h0_hbm.at[idx_v.at[j]],
                        rows_v.at[pl.ds(j * CHUNK, CHUNK)],
                        sem,
                    )
                )
            for d in descs:
                d.wait()
            return carry

        lax.fori_loop(0, NCHUNK // FIRE, group, 0)
        pltpu.sync_copy(rows_v, src_hbm.at[pl.ds(wid * PW_PAD, PW_PAD)])

    return gather_kernel(row_idx, h0f)


def _sc_scatter(msg, col_idx):
    mesh = plsc.VectorSubcoreMesh(core_axis_name="c", subcore_axis_name="s")

    @functools.partial(
        pl.kernel,
        out_type=(
            jax.ShapeDtypeStruct((NC * N_PAD, M), jnp.float32),
            jax.ShapeDtypeStruct((NC * N_PAD, M), jnp.float32),
        ),
        mesh=mesh,
        compiler_params=pltpu.CompilerParams(use_tc_tiling_on_sc=False),
        scratch_types=[
            pltpu.VMEM((NCHUNK, CHUNK), jnp.int32),
            pltpu.VMEM((PW_PAD, M), jnp.float32),
            pltpu.VMEM((CHUNK, M), jnp.float32),
            pltpu.VMEM((ZROWS, M), jnp.float32),
            pltpu.VMEM_SHARED((N_PAD, M), jnp.float32),
            pltpu.VMEM_SHARED((N_PAD, M), jnp.float32),
        ],
    )
    def scatter_kernel(msg_hbm, col_hbm, seg_hbm, cnt_hbm,
                       idx_v, msg_v, ones_v, zz_v, seg_sp, cnt_sp):
        cid = lax.axis_index("c")
        sid = lax.axis_index("s")
        wid = sid * NC + cid

        zrow = jnp.zeros((M,), jnp.float32)

        def zbody(i, c):
            zz_v[i, :] = zrow
            return c

        lax.fori_loop(0, ZROWS, zbody, 0)

        orow = jnp.ones((M,), jnp.float32)

        def obody(i, c):
            ones_v[i, :] = orow
            return c

        lax.fori_loop(0, CHUNK, obody, 0)

        rbase = sid * ZROWS
        pltpu.sync_copy(zz_v, seg_sp.at[pl.ds(rbase, ZROWS)])
        pltpu.sync_copy(zz_v, cnt_sp.at[pl.ds(rbase, ZROWS)])
        pltpu.sync_copy(col_hbm.at[wid], idx_v)
        pltpu.sync_copy(msg_hbm.at[pl.ds(wid * PW_PAD, PW_PAD)], msg_v)
        plsc.subcore_barrier()

        def sbody(j, c):
            pltpu.sync_copy(msg_v.at[pl.ds(j * CHUNK, CHUNK)],
                            seg_sp.at[idx_v.at[j]], add=True)
            pltpu.sync_copy(ones_v, cnt_sp.at[idx_v.at[j]], add=True)
            return c

        lax.fori_loop(0, NCHUNK, sbody, 0)
        plsc.subcore_barrier()

        pltpu.sync_copy(seg_sp.at[pl.ds(rbase, ZROWS)], zz_v)
        pltpu.sync_copy(zz_v, seg_hbm.at[pl.ds(cid * N_PAD + rbase, ZROWS)])
        pltpu.sync_copy(cnt_sp.at[pl.ds(rbase, ZROWS)], zz_v)
        pltpu.sync_copy(zz_v, cnt_hbm.at[pl.ds(cid * N_PAD + rbase, ZROWS)])

    return scatter_kernel(msg, col_idx)


def _ln_relu(x, g, b):
    mu = jnp.mean(x, axis=-1, keepdims=True)
    var = jnp.mean((x - mu) * (x - mu), axis=-1, keepdims=True)
    y = (x - mu) * lax.rsqrt(var + 1e-5) * g + b
    return jnp.maximum(y, 0.0)


def _tc_messages(efp, rp, basp, src, W1a, W1b, b1, g1, be1,
                 W2, b2, g2, be2, W3, b3, T, S):
    def body(ef_ref, r_ref, bas_ref, src_ref, w1a, w1b, b1r, g1r, be1r,
             w2, b2r, g2r, be2r, w3, b3r, t, s, out_ref):
        x = jnp.dot(ef_ref[...], w1a[...], preferred_element_type=jnp.float32)
        x = x + r_ref[...] * w1b[...] + b1r[...]
        x = _ln_relu(x, g1r[...], be1r[...])
        x = jnp.dot(x, w2[...], preferred_element_type=jnp.float32) + b2r[...]
        x = _ln_relu(x, g2r[...], be2r[...])
        y = jnp.dot(x, w3[...], preferred_element_type=jnp.float32) + b3r[...]
        srep = jnp.dot(src_ref[...], t[...], preferred_element_type=jnp.float32)
        m = jnp.dot(y * srep, s[...], preferred_element_type=jnp.float32)
        out_ref[...] = m * bas_ref[...]

    eb16 = pl.BlockSpec((EB, M), lambda i: (i, 0))
    eb1 = pl.BlockSpec((EB, 1), lambda i: (i, 0))

    def wspec(a):
        return pl.BlockSpec(a.shape, lambda i: (0,) * a.ndim)

    weights = (W1a, W1b, b1, g1, be1, W2, b2, g2, be2, W3, b3, T, S)
    return pl.pallas_call(
        body,
        grid=(E_PAD // EB,),
        in_specs=[eb16, eb1, eb1, eb16] + [wspec(w) for w in weights],
        out_specs=eb16,
        out_shape=jax.ShapeDtypeStruct((E_PAD, M), jnp.float32),
    )(efp, rp, basp, src, *weights)


def _tc_finalize(seg2, cnt2, h0p, wst):
    NB = 4
    RB = N_PAD // NB

    def body(seg_ref, cnt_ref, h0_ref, w_ref, out_ref):
        sg = jnp.sum(seg_ref[...], axis=0)
        ct = jnp.sum(cnt_ref[...], axis=0)
        sf = jnp.dot(h0_ref[...], w_ref[...], preferred_element_type=jnp.float32)
        out_ref[...] = sg / jnp.maximum(ct, 1.0) + jnp.where(ct > 0.0, sf, 0.0)

    seg_spec = pl.BlockSpec((NC, RB, M), lambda i: (0, i, 0))
    row_spec = pl.BlockSpec((RB, M), lambda i: (i, 0))
    w_spec = pl.BlockSpec((M, M), lambda i: (0, 0))
    return pl.pallas_call(
        body,
        grid=(NB,),
        in_specs=[seg_spec, seg_spec, row_spec, w_spec],
        out_specs=row_spec,
        out_shape=jax.ShapeDtypeStruct((N_PAD, M), jnp.float32),
    )(seg2, cnt2, h0p, wst)


def kernel(h0, r, basis_00, edge_index, edge_feat,
           W1, b1, g1, be1, W2, b2, g2, be2, W3, b3, W_self):
    h0f = h0.reshape(N_NODES, M)
    row = edge_index[0]
    col = edge_index[1]
    pad = E_PAD - N_EDGES
    rowp = jnp.concatenate([row, jnp.zeros((pad,), jnp.int32)])
    rowp = rowp.reshape(NW, NCHUNK, CHUNK)
    colp = jnp.concatenate([col, jnp.full((pad,), N_NODES, jnp.int32)])
    colp = colp.reshape(NW, NCHUNK, CHUNK)
    efp = jnp.concatenate([edge_feat, jnp.zeros((pad, M), jnp.float32)])
    rp = jnp.concatenate([r, jnp.zeros((pad, 1), jnp.float32)])
    basp = jnp.concatenate(
        [basis_00.reshape(N_EDGES, 1), jnp.zeros((pad, 1), jnp.float32)])

    src = _sc_gather(rowp, h0f)

    k = jnp.arange(OUT3)
    T = (k[None, :] % M == jnp.arange(M)[:, None]).astype(jnp.float32)
    S = (k[:, None] // M == jnp.arange(M)[None, :]).astype(jnp.float32)
    msg = _tc_messages(
        efp, rp, basp, src,
        W1[:M], W1[M:], b1.reshape(1, MID), g1.reshape(1, MID),
        be1.reshape(1, MID), W2, b2.reshape(1, MID), g2.reshape(1, MID),
        be2.reshape(1, MID), W3, b3.reshape(1, OUT3), T, S)

    seg2, cnt2 = _sc_scatter(msg, colp)

    h0p = jnp.concatenate(
        [h0f, jnp.zeros((N_PAD - N_NODES, M), jnp.float32)])
    outp = _tc_finalize(seg2.reshape(NC, N_PAD, M),
                        cnt2.reshape(NC, N_PAD, M), h0p, W_self[0].T)
    return outp[:N_NODES].reshape(N_NODES, M, 1)


# speedup vs baseline: 4.3847x; 1.6609x over previous
import functools

import jax
import jax.numpy as jnp
from jax import lax
from jax.experimental import pallas as pl
from jax.experimental.pallas import tpu as pltpu
from jax.experimental.pallas import tpu_sc as plsc

N_NODES = 10000
N_EDGES = 160000
M = 16
MID = 32
OUT3 = 256

NC = 2
NS = 16
NW = NC * NS
CHUNK = 128
NCH = N_EDGES // CHUNK
TFULL = NCH // NW
NEXTRA = NCH - TFULL * NW
MAXCH = TFULL + 1
FIRE = 8
N_PAD = 10240
ZROWS = N_PAD // NS

P = 4
EB = 3200
EBP = EB // P
E4 = N_EDGES // P


def _worker_base(wid):
    return wid * TFULL + jnp.minimum(wid, NEXTRA)


def _sc_gather(row2, h0f):
    mesh = plsc.VectorSubcoreMesh(core_axis_name="c", subcore_axis_name="s")

    @functools.partial(
        pl.kernel,
        out_type=jax.ShapeDtypeStruct((N_EDGES, M), jnp.float32),
        mesh=mesh,
        compiler_params=pltpu.CompilerParams(use_tc_tiling_on_sc=False),
        scratch_types=[
            pltpu.VMEM((MAXCH, CHUNK), jnp.int32),
            pltpu.VMEM((MAXCH * CHUNK, M), jnp.float32),
            pltpu.SemaphoreType.DMA,
        ],
    )
    def gather_kernel(row_hbm, h0_hbm, src_hbm, idx_v, rows_v, sem):
        cid = lax.axis_index("c")
        sid = lax.axis_index("s")
        wid = sid * NC + cid
        base = _worker_base(wid)
        pltpu.sync_copy(row_hbm.at[pl.ds(base, TFULL)],
                        idx_v.at[pl.ds(0, TFULL)])

        def group(g, carry):
            t0 = g * FIRE
            descs = []
            for k in range(FIRE):
                t = t0 + k
                descs.append(pltpu.async_copy(
                    h0_hbm.at[idx_v.at[t]],
                    rows_v.at[pl.ds(t * CHUNK, CHUNK)],
                    sem))
            for d in descs:
                d.wait()
            return carry

        lax.fori_loop(0, TFULL // FIRE, group, 0)
        tail = []
        for t in range(FIRE * (TFULL // FIRE), TFULL):
            tail.append(pltpu.async_copy(
                h0_hbm.at[idx_v.at[t]],
                rows_v.at[pl.ds(t * CHUNK, CHUNK)],
                sem))
        for d in tail:
            d.wait()
        pltpu.sync_copy(rows_v.at[pl.ds(0, TFULL * CHUNK)],
                        src_hbm.at[pl.ds(base * CHUNK, TFULL * CHUNK)])

        @pl.when(wid < NEXTRA)
        def _():
            pltpu.sync_copy(row_hbm.at[base + TFULL], idx_v.at[TFULL])
            pltpu.async_copy(
                h0_hbm.at[idx_v.at[TFULL]],
                rows_v.at[pl.ds(TFULL * CHUNK, CHUNK)], sem).wait()
            pltpu.sync_copy(
                rows_v.at[pl.ds(TFULL * CHUNK, CHUNK)],
                src_hbm.at[pl.ds((base + TFULL) * CHUNK, CHUNK)])

    return gather_kernel(row2, h0f)


def _sc_scatter(msg, col2):
    mesh = plsc.VectorSubcoreMesh(core_axis_name="c", subcore_axis_name="s")

    @functools.partial(
        pl.kernel,
        out_type=(
            jax.ShapeDtypeStruct((NC * N_PAD, M), jnp.float32),
            jax.ShapeDtypeStruct((NC * N_PAD, M), jnp.float32),
        ),
        mesh=mesh,
        compiler_params=pltpu.CompilerParams(use_tc_tiling_on_sc=False),
        scratch_types=[
            pltpu.VMEM((MAXCH, CHUNK), jnp.int32),
            pltpu.VMEM((MAXCH * CHUNK, M), jnp.float32),
            pltpu.VMEM((CHUNK, M), jnp.float32),
            pltpu.VMEM((ZROWS, M), jnp.float32),
            pltpu.VMEM_SHARED((N_PAD, M), jnp.float32),
            pltpu.VMEM_SHARED((N_PAD, M), jnp.float32),
            pltpu.SemaphoreType.DMA,
        ],
    )
    def scatter_kernel(msg_hbm, col_hbm, seg_hbm, cnt_hbm,
                       idx_v, msg_v, ones_v, zz_v, seg_sp, cnt_sp, sem):
        cid = lax.axis_index("c")
        sid = lax.axis_index("s")
        wid = sid * NC + cid
        base = _worker_base(wid)

        zrow = jnp.zeros((M,), jnp.float32)

        def zbody(i, c):
            zz_v[i, :] = zrow
            return c

        lax.fori_loop(0, ZROWS, zbody, 0)

        orow = jnp.ones((M,), jnp.float32)

        def obody(i, c):
            ones_v[i, :] = orow
            return c

        lax.fori_loop(0, CHUNK, obody, 0)

        rbase = sid * ZROWS
        pltpu.sync_copy(zz_v, seg_sp.at[pl.ds(rbase, ZROWS)])
        pltpu.sync_copy(zz_v, cnt_sp.at[pl.ds(rbase, ZROWS)])
        pltpu.sync_copy(col_hbm.at[pl.ds(base, TFULL)],
                        idx_v.at[pl.ds(0, TFULL)])
        pltpu.sync_copy(msg_hbm.at[pl.ds(base * CHUNK, TFULL * CHUNK)],
                        msg_v.at[pl.ds(0, TFULL * CHUNK)])

        @pl.when(wid < NEXTRA)
        def _():
            pltpu.sync_copy(col_hbm.at[base + TFULL], idx_v.at[TFULL])
            pltpu.sync_copy(
                msg_hbm.at[pl.ds((base + TFULL) * CHUNK, CHUNK)],
                msg_v.at[pl.ds(TFULL * CHUNK, CHUNK)])

        plsc.subcore_barrier()

        def sgroup(g, carry):
            t0 = g * FIRE
            descs = []
            for k in range(FIRE):
                t = t0 + k
                descs.append(pltpu.async_copy(
                    msg_v.at[pl.ds(t * CHUNK, CHUNK)],
                    seg_sp.at[idx_v.at[t]], sem, add=True))
                descs.append(pltpu.async_copy(
                    ones_v, cnt_sp.at[idx_v.at[t]], sem, add=True))
            for d in descs:
                d.wait()
            return carry

        lax.fori_loop(0, TFULL // FIRE, sgroup, 0)
        tail = []
        for t in range(FIRE * (TFULL // FIRE), TFULL):
            tail.append(pltpu.async_copy(
                msg_v.at[pl.ds(t * CHUNK, CHUNK)],
                seg_sp.at[idx_v.at[t]], sem, add=True))
            tail.append(pltpu.async_copy(
                ones_v, cnt_sp.at[idx_v.at[t]], sem, add=True))
        for d in tail:
            d.wait()

        @pl.when(wid < NEXTRA)
        def _():
            pltpu.async_copy(
                msg_v.at[pl.ds(TFULL * CHUNK, CHUNK)],
                seg_sp.at[idx_v.at[TFULL]], sem, add=True).wait()
            pltpu.async_copy(
                ones_v, cnt_sp.at[idx_v.at[TFULL]], sem, add=True).wait()

        plsc.subcore_barrier()

        pltpu.sync_copy(seg_sp.at[pl.ds(rbase, ZROWS)], zz_v)
        pltpu.sync_copy(zz_v, seg_hbm.at[pl.ds(cid * N_PAD + rbase, ZROWS)])
        pltpu.sync_copy(cnt_sp.at[pl.ds(rbase, ZROWS)], zz_v)
        pltpu.sync_copy(zz_v, cnt_hbm.at[pl.ds(cid * N_PAD + rbase, ZROWS)])

    return scatter_kernel(msg, col2)


def _ln_relu_packed(x, gt, bet, gsum, ubc):
    mu = jnp.dot(x, gsum, preferred_element_type=jnp.float32)
    m2 = jnp.dot(x * x, gsum, preferred_element_type=jnp.float32)
    var = m2 - mu * mu
    rs = lax.rsqrt(var + 1e-5)
    scale = jnp.dot(rs, ubc, preferred_element_type=jnp.float32)
    shift = jnp.dot(mu * rs, ubc, preferred_element_type=jnp.float32)
    return jnp.maximum((x * scale - shift) * gt + bet, 0.0)


def _tc_messages(efP, rP, basP, srcP, consts):
    def body(ef_ref, r_ref, bas_ref, src_ref, w1bd, r1bd, b1t, g1t, be1t,
             w2bd, b2t, g2t, be2t, w3bd, b3t, tbd, sbd, gsum, ubc, bmsg,
             out_ref):
        x = jnp.dot(ef_ref[...], w1bd[...], preferred_element_type=jnp.float32)
        x = x + jnp.dot(r_ref[...], r1bd[...],
                        preferred_element_type=jnp.float32) + b1t[...]
        x = _ln_relu_packed(x, g1t[...], be1t[...], gsum[...], ubc[...])
        x = jnp.dot(x, w2bd[...], preferred_element_type=jnp.float32) + b2t[...]
        x = _ln_relu_packed(x, g2t[...], be2t[...], gsum[...], ubc[...])
        y = jnp.dot(x, w3bd[...], preferred_element_type=jnp.float32) + b3t[...]
        srep = jnp.dot(src_ref[...], tbd[...],
                       preferred_element_type=jnp.float32)
        m = jnp.dot(y * srep, sbd[...], preferred_element_type=jnp.float32)
        out_ref[...] = m * jnp.dot(bas_ref[...], bmsg[...],
                                   preferred_element_type=jnp.float32)

    eb64 = pl.BlockSpec((EBP, P * M), lambda i: (i, 0))
    eb4 = pl.BlockSpec((EBP, P), lambda i: (i, 0))

    def wspec(a):
        return pl.BlockSpec(a.shape, lambda i: (0,) * a.ndim)

    return pl.pallas_call(
        body,
        grid=(E4 // EBP,),
        in_specs=[eb64, eb4, eb4, eb64] + [wspec(w) for w in consts],
        out_specs=eb64,
        out_shape=jax.ShapeDtypeStruct((E4, P * M), jnp.float32),
    )(efP, rP, basP, srcP, *consts)


def _tc_finalize(seg2, cnt2, h0f, wst):
    NB = 5
    RB = N_NODES // NB

    def body(seg_ref, cnt_ref, h0_ref, w_ref, out_ref):
        sg = jnp.sum(seg_ref[...], axis=0)
        ct = jnp.sum(cnt_ref[...], axis=0)
        sf = jnp.dot(h0_ref[...], w_ref[...], preferred_element_type=jnp.float32)
        out_ref[...] = sg / jnp.maximum(ct, 1.0) + jnp.where(ct > 0.0, sf, 0.0)

    seg_spec = pl.BlockSpec((NC, RB, M), lambda i: (0, i, 0))
    row_spec = pl.BlockSpec((RB, M), lambda i: (i, 0))
    w_spec = pl.BlockSpec((M, M), lambda i: (0, 0))
    return pl.pallas_call(
        body,
        grid=(NB,),
        in_specs=[seg_spec, seg_spec, row_spec, w_spec],
        out_specs=row_spec,
        out_shape=jax.ShapeDtypeStruct((N_NODES, M), jnp.float32),
    )(seg2, cnt2, h0f, wst)


def kernel(h0, r, basis_00, edge_index, edge_feat,
           W1, b1, g1, be1, W2, b2, g2, be2, W3, b3, W_self):
    f32 = jnp.float32
    h0f = h0.reshape(N_NODES, M)
    row2 = edge_index[0].reshape(NCH, CHUNK)
    col2 = edge_index[1].reshape(NCH, CHUNK)

    src = _sc_gather(row2, h0f)

    eye4 = jnp.eye(P, dtype=f32)
    k = jnp.arange(OUT3)
    T = jnp.tile(jnp.eye(M, dtype=f32), (1, M))
    S = (k[:, None] // M == jnp.arange(M)[None, :]).astype(f32)
    consts = (
        jnp.kron(eye4, W1[:M]),
        jnp.kron(eye4, W1[M:]),
        jnp.tile(b1, P).reshape(1, P * MID),
        jnp.tile(g1, P).reshape(1, P * MID),
        jnp.tile(be1, P).reshape(1, P * MID),
        jnp.kron(eye4, W2),
        jnp.tile(b2, P).reshape(1, P * MID),
        jnp.tile(g2, P).reshape(1, P * MID),
        jnp.tile(be2, P).reshape(1, P * MID),
        jnp.kron(eye4, W3),
        jnp.tile(b3, P).reshape(1, P * OUT3),
        jnp.kron(eye4, T),
        jnp.kron(eye4, S),
        jnp.kron(eye4, jnp.full((MID, 1), 1.0 / MID, f32)),
        jnp.kron(eye4, jnp.ones((1, MID), f32)),
        jnp.kron(eye4, jnp.ones((1, M), f32)),
    )
    msg4 = _tc_messages(
        edge_feat.reshape(E4, P * M), r.reshape(E4, P),
        basis_00.reshape(E4, P), src.reshape(E4, P * M), consts)

    seg2, cnt2 = _sc_scatter(msg4.reshape(N_EDGES, M), col2)

    outp = _tc_finalize(seg2.reshape(NC, N_PAD, M),
                        cnt2.reshape(NC, N_PAD, M), h0f, W_self[0].T)
    return outp.reshape(N_NODES, M, 1)


# speedup vs baseline: 4.6329x; 1.0566x over previous
import functools

import jax
import jax.numpy as jnp
from jax import lax
from jax.experimental import pallas as pl
from jax.experimental.pallas import tpu as pltpu
from jax.experimental.pallas import tpu_sc as plsc

N_NODES = 10000
N_EDGES = 160000
M = 16
MID = 32
OUT3 = 256

NC = 2
NS = 16
NW = NC * NS
CHUNK = 128
NCH = N_EDGES // CHUNK
TFULL = NCH // NW
NEXTRA = NCH - TFULL * NW
MAXCH = TFULL + 1
FIRE = 8
N_PAD = 10240
ZROWS = N_PAD // NS

P = 4
EB = 6400
EBP = EB // P
E4 = N_EDGES // P


def _worker_base(wid):
    return wid * TFULL + jnp.minimum(wid, NEXTRA)


def _sc_gather(ei3, h0f):
    mesh = plsc.VectorSubcoreMesh(core_axis_name="c", subcore_axis_name="s")

    @functools.partial(
        pl.kernel,
        out_type=jax.ShapeDtypeStruct((N_EDGES, M), jnp.float32),
        mesh=mesh,
        compiler_params=pltpu.CompilerParams(use_tc_tiling_on_sc=False),
        scratch_types=[
            pltpu.VMEM((MAXCH, CHUNK), jnp.int32),
            pltpu.VMEM((MAXCH * CHUNK, M), jnp.float32),
            pltpu.SemaphoreType.DMA,
        ],
    )
    def gather_kernel(ei_hbm, h0_hbm, src_hbm, idx_v, rows_v, sem):
        cid = lax.axis_index("c")
        sid = lax.axis_index("s")
        wid = sid * NC + cid
        base = _worker_base(wid)
        row_hbm = ei_hbm.at[0]
        pltpu.sync_copy(row_hbm.at[pl.ds(base, TFULL)],
                        idx_v.at[pl.ds(0, TFULL)])

        def group(g, carry):
            t0 = g * FIRE
            descs = []
            for k in range(FIRE):
                t = t0 + k
                descs.append(pltpu.async_copy(
                    h0_hbm.at[idx_v.at[t]],
                    rows_v.at[pl.ds(t * CHUNK, CHUNK)],
                    sem))
            for d in descs:
                d.wait()
            return carry

        lax.fori_loop(0, TFULL // FIRE, group, 0)
        tail = []
        for t in range(FIRE * (TFULL // FIRE), TFULL):
            tail.append(pltpu.async_copy(
                h0_hbm.at[idx_v.at[t]],
                rows_v.at[pl.ds(t * CHUNK, CHUNK)],
                sem))
        for d in tail:
            d.wait()
        pltpu.sync_copy(rows_v.at[pl.ds(0, TFULL * CHUNK)],
                        src_hbm.at[pl.ds(base * CHUNK, TFULL * CHUNK)])

        @pl.when(wid < NEXTRA)
        def _():
            pltpu.sync_copy(row_hbm.at[base + TFULL], idx_v.at[TFULL])
            pltpu.async_copy(
                h0_hbm.at[idx_v.at[TFULL]],
                rows_v.at[pl.ds(TFULL * CHUNK, CHUNK)], sem).wait()
            pltpu.sync_copy(
                rows_v.at[pl.ds(TFULL * CHUNK, CHUNK)],
                src_hbm.at[pl.ds((base + TFULL) * CHUNK, CHUNK)])

    return gather_kernel(ei3, h0f)


def _sc_scatter(msg, ei3):
    mesh = plsc.VectorSubcoreMesh(core_axis_name="c", subcore_axis_name="s")

    @functools.partial(
        pl.kernel,
        out_type=(
            jax.ShapeDtypeStruct((NC * N_PAD, M), jnp.float32),
            jax.ShapeDtypeStruct((NC * N_PAD, M), jnp.float32),
        ),
        mesh=mesh,
        compiler_params=pltpu.CompilerParams(use_tc_tiling_on_sc=False),
        scratch_types=[
            pltpu.VMEM((MAXCH, CHUNK), jnp.int32),
            pltpu.VMEM((MAXCH * CHUNK, M), jnp.float32),
            pltpu.VMEM((CHUNK, M), jnp.float32),
            pltpu.VMEM((ZROWS, M), jnp.float32),
            pltpu.VMEM_SHARED((N_PAD, M), jnp.float32),
            pltpu.VMEM_SHARED((N_PAD, M), jnp.float32),
            pltpu.SemaphoreType.DMA,
        ],
    )
    def scatter_kernel(msg_hbm, ei_hbm, seg_hbm, cnt_hbm,
                       idx_v, msg_v, ones_v, zz_v, seg_sp, cnt_sp, sem):
        cid = lax.axis_index("c")
        sid = lax.axis_index("s")
        wid = sid * NC + cid
        base = _worker_base(wid)
        col_hbm = ei_hbm.at[1]

        zrow = jnp.zeros((M,), jnp.float32)

        def zbody(i, c):
            zz_v[i, :] = zrow
            return c

        lax.fori_loop(0, ZROWS, zbody, 0)

        orow = jnp.ones((M,), jnp.float32)

        def obody(i, c):
            ones_v[i, :] = orow
            return c

        lax.fori_loop(0, CHUNK, obody, 0)

        rbase = sid * ZROWS
        pltpu.sync_copy(zz_v, seg_sp.at[pl.ds(rbase, ZROWS)])
        pltpu.sync_copy(zz_v, cnt_sp.at[pl.ds(rbase, ZROWS)])
        pltpu.sync_copy(col_hbm.at[pl.ds(base, TFULL)],
                        idx_v.at[pl.ds(0, TFULL)])
        pltpu.sync_copy(msg_hbm.at[pl.ds(base * CHUNK, TFULL * CHUNK)],
                        msg_v.at[pl.ds(0, TFULL * CHUNK)])

        @pl.when(wid < NEXTRA)
        def _():
            pltpu.sync_copy(col_hbm.at[base + TFULL], idx_v.at[TFULL])
            pltpu.sync_copy(
                msg_hbm.at[pl.ds((base + TFULL) * CHUNK, CHUNK)],
                msg_v.at[pl.ds(TFULL * CHUNK, CHUNK)])

        plsc.subcore_barrier()

        def sgroup(g, carry):
            t0 = g * FIRE
            descs = []
            for k in range(FIRE):
                t = t0 + k
                descs.append(pltpu.async_copy(
                    msg_v.at[pl.ds(t * CHUNK, CHUNK)],
                    seg_sp.at[idx_v.at[t]], sem, add=True))
                descs.append(pltpu.async_copy(
                    ones_v, cnt_sp.at[idx_v.at[t]], sem, add=True))
            for d in descs:
                d.wait()
            return carry

        lax.fori_loop(0, TFULL // FIRE, sgroup, 0)
        tail = []
        for t in range(FIRE * (TFULL // FIRE), TFULL):
            tail.append(pltpu.async_copy(
                msg_v.at[pl.ds(t * CHUNK, CHUNK)],
                seg_sp.at[idx_v.at[t]], sem, add=True))
            tail.append(pltpu.async_copy(
                ones_v, cnt_sp.at[idx_v.at[t]], sem, add=True))
        for d in tail:
            d.wait()

        @pl.when(wid < NEXTRA)
        def _():
            pltpu.async_copy(
                msg_v.at[pl.ds(TFULL * CHUNK, CHUNK)],
                seg_sp.at[idx_v.at[TFULL]], sem, add=True).wait()
            pltpu.async_copy(
                ones_v, cnt_sp.at[idx_v.at[TFULL]], sem, add=True).wait()

        plsc.subcore_barrier()

        pltpu.sync_copy(seg_sp.at[pl.ds(rbase, ZROWS)], zz_v)
        pltpu.sync_copy(zz_v, seg_hbm.at[pl.ds(cid * N_PAD + rbase, ZROWS)])
        pltpu.sync_copy(cnt_sp.at[pl.ds(rbase, ZROWS)], zz_v)
        pltpu.sync_copy(zz_v, cnt_hbm.at[pl.ds(cid * N_PAD + rbase, ZROWS)])

    return scatter_kernel(msg, ei3)


def _ln_relu_packed(x, gt, bet, gsum, ubc):
    mu = jnp.dot(x, gsum, preferred_element_type=jnp.float32)
    m2 = jnp.dot(x * x, gsum, preferred_element_type=jnp.float32)
    var = m2 - mu * mu
    rs = lax.rsqrt(var + 1e-5)
    scale = jnp.dot(rs, ubc, preferred_element_type=jnp.float32)
    shift = jnp.dot(mu * rs, ubc, preferred_element_type=jnp.float32)
    return jnp.maximum((x * scale - shift) * gt + bet, 0.0)


def _tc_messages(efP, rP, basP, srcP, consts):
    def body(ef_ref, r_ref, bas_ref, src_ref, w1bd, r1bd, b1t, g1t, be1t,
             w2bd, b2t, g2t, be2t, w3bd, b3t, tbd, sbd, gsum, ubc, bmsg,
             out_ref):
        x = jnp.dot(ef_ref[...], w1bd[...], preferred_element_type=jnp.float32)
        x = x + jnp.dot(r_ref[...], r1bd[...],
                        preferred_element_type=jnp.float32) + b1t[...]
        x = _ln_relu_packed(x, g1t[...], be1t[...], gsum[...], ubc[...])
        x = jnp.dot(x, w2bd[...], preferred_element_type=jnp.float32) + b2t[...]
        x = _ln_relu_packed(x, g2t[...], be2t[...], gsum[...], ubc[...])
        y = jnp.dot(x, w3bd[...], preferred_element_type=jnp.float32) + b3t[...]
        srep = jnp.dot(src_ref[...], tbd[...],
                       preferred_element_type=jnp.float32)
        m = jnp.dot(y * srep, sbd[...], preferred_element_type=jnp.float32)
        out_ref[...] = m * jnp.dot(bas_ref[...], bmsg[...],
                                   preferred_element_type=jnp.float32)

    eb64 = pl.BlockSpec((EBP, P * M), lambda i: (i, 0))
    eb4 = pl.BlockSpec((EBP, P), lambda i: (i, 0))

    def wspec(a):
        return pl.BlockSpec(a.shape, lambda i: (0,) * a.ndim)

    return pl.pallas_call(
        body,
        grid=(E4 // EBP,),
        in_specs=[eb64, eb4, eb4, eb64] + [wspec(w) for w in consts],
        out_specs=eb64,
        out_shape=jax.ShapeDtypeStruct((E4, P * M), jnp.float32),
    )(efP, rP, basP, srcP, *consts)


def _tc_finalize(seg2, cnt2, h0f, wst):
    NB = 5
    RB = N_NODES // NB

    def body(seg_ref, cnt_ref, h0_ref, w_ref, out_ref):
        sg = jnp.sum(seg_ref[...], axis=0)
        ct = jnp.sum(cnt_ref[...], axis=0)
        sf = jnp.dot(h0_ref[...], w_ref[...], preferred_element_type=jnp.float32)
        out_ref[...] = sg / jnp.maximum(ct, 1.0) + jnp.where(ct > 0.0, sf, 0.0)

    seg_spec = pl.BlockSpec((NC, RB, M), lambda i: (0, i, 0))
    row_spec = pl.BlockSpec((RB, M), lambda i: (i, 0))
    w_spec = pl.BlockSpec((M, M), lambda i: (0, 0))
    return pl.pallas_call(
        body,
        grid=(NB,),
        in_specs=[seg_spec, seg_spec, row_spec, w_spec],
        out_specs=row_spec,
        out_shape=jax.ShapeDtypeStruct((N_NODES, M), jnp.float32),
    )(seg2, cnt2, h0f, wst)


def kernel(h0, r, basis_00, edge_index, edge_feat,
           W1, b1, g1, be1, W2, b2, g2, be2, W3, b3, W_self):
    f32 = jnp.float32
    h0f = h0.reshape(N_NODES, M)
    ei3 = edge_index.reshape(2, NCH, CHUNK)

    src = _sc_gather(ei3, h0f)

    eye4 = jnp.eye(P, dtype=f32)
    k = jnp.arange(OUT3)
    T = jnp.tile(jnp.eye(M, dtype=f32), (1, M))
    S = (k[:, None] // M == jnp.arange(M)[None, :]).astype(f32)
    gsum = jnp.kron(eye4, jnp.full((MID, 1), 1.0 / MID, f32))
    ubc = jnp.kron(eye4, jnp.ones((1, MID), f32))
    consts = (
        jnp.kron(eye4, W1[:M]),
        jnp.kron(eye4, W1[M:]),
        jnp.tile(b1, P).reshape(1, P * MID),
        jnp.tile(g1, P).reshape(1, P * MID),
        jnp.tile(be1, P).reshape(1, P * MID),
        jnp.kron(eye4, W2),
        jnp.tile(b2, P).reshape(1, P * MID),
        jnp.tile(g2, P).reshape(1, P * MID),
        jnp.tile(be2, P).reshape(1, P * MID),
        jnp.kron(eye4, W3),
        jnp.tile(b3, P).reshape(1, P * OUT3),
        jnp.kron(eye4, T),
        jnp.kron(eye4, S),
        gsum,
        ubc,
        jnp.kron(eye4, jnp.ones((1, M), f32)),
    )
    msg4 = _tc_messages(
        edge_feat.reshape(E4, P * M), r.reshape(E4, P),
        basis_00.reshape(E4, P), src.reshape(E4, P * M), consts)

    seg2, cnt2 = _sc_scatter(msg4.reshape(N_EDGES, M), ei3)

    outp = _tc_finalize(seg2.reshape(NC, N_PAD, M),
                        cnt2.reshape(NC, N_PAD, M), h0f, W_self[0].T)
    return outp.reshape(N_NODES, M, 1)


# speedup vs baseline: 5.4370x; 1.1736x over previous
import functools

import jax
import jax.numpy as jnp
from jax import lax
from jax.experimental import pallas as pl
from jax.experimental.pallas import tpu as pltpu
from jax.experimental.pallas import tpu_sc as plsc

N_NODES = 10000
N_EDGES = 160000
M = 16
MID = 32
OUT3 = 256

NC = 2
NS = 16
NW = NC * NS
CHUNK = 128
NCH = N_EDGES // CHUNK
TFULL = NCH // NW
NEXTRA = NCH - TFULL * NW
MAXCH = TFULL + 1
FIRE = 8
N_PAD = 10240
ZROWS = N_PAD // NS

P = 8
EB = 6400
EBP = EB // P
E8 = N_EDGES // P
N8 = N_NODES // P
NP8 = N_PAD * M // (P * M)


def _worker_base(wid):
    return wid * TFULL + jnp.minimum(wid, NEXTRA)


def _sc_gather(ei3, h0f):
    mesh = plsc.VectorSubcoreMesh(core_axis_name="c", subcore_axis_name="s")

    @functools.partial(
        pl.kernel,
        out_type=jax.ShapeDtypeStruct((N_EDGES, M), jnp.float32),
        mesh=mesh,
        compiler_params=pltpu.CompilerParams(use_tc_tiling_on_sc=False),
        scratch_types=[
            pltpu.VMEM((MAXCH, CHUNK), jnp.int32),
            pltpu.VMEM((MAXCH * CHUNK, M), jnp.float32),
            pltpu.SemaphoreType.DMA,
        ],
    )
    def gather_kernel(ei_hbm, h0_hbm, src_hbm, idx_v, rows_v, sem):
        cid = lax.axis_index("c")
        sid = lax.axis_index("s")
        wid = sid * NC + cid
        base = _worker_base(wid)
        row_hbm = ei_hbm.at[0]
        pltpu.sync_copy(row_hbm.at[pl.ds(base, TFULL)],
                        idx_v.at[pl.ds(0, TFULL)])

        def group(g, carry):
            t0 = g * FIRE
            descs = []
            for k in range(FIRE):
                t = t0 + k
                descs.append(pltpu.async_copy(
                    h0_hbm.at[idx_v.at[t]],
                    rows_v.at[pl.ds(t * CHUNK, CHUNK)],
                    sem))
            for d in descs:
                d.wait()
            return carry

        lax.fori_loop(0, TFULL // FIRE, group, 0)
        tail = []
        for t in range(FIRE * (TFULL // FIRE), TFULL):
            tail.append(pltpu.async_copy(
                h0_hbm.at[idx_v.at[t]],
                rows_v.at[pl.ds(t * CHUNK, CHUNK)],
                sem))
        for d in tail:
            d.wait()
        pltpu.sync_copy(rows_v.at[pl.ds(0, TFULL * CHUNK)],
                        src_hbm.at[pl.ds(base * CHUNK, TFULL * CHUNK)])

        @pl.when(wid < NEXTRA)
        def _():
            pltpu.sync_copy(row_hbm.at[base + TFULL], idx_v.at[TFULL])
            pltpu.async_copy(
                h0_hbm.at[idx_v.at[TFULL]],
                rows_v.at[pl.ds(TFULL * CHUNK, CHUNK)], sem).wait()
            pltpu.sync_copy(
                rows_v.at[pl.ds(TFULL * CHUNK, CHUNK)],
                src_hbm.at[pl.ds((base + TFULL) * CHUNK, CHUNK)])

    return gather_kernel(ei3, h0f)


def _sc_scatter(msg, ei3):
    mesh = plsc.VectorSubcoreMesh(core_axis_name="c", subcore_axis_name="s")

    @functools.partial(
        pl.kernel,
        out_type=(
            jax.ShapeDtypeStruct((N_PAD, M), jnp.float32),
            jax.ShapeDtypeStruct((N_PAD, M), jnp.float32),
            jax.ShapeDtypeStruct((N_PAD, M), jnp.float32),
            jax.ShapeDtypeStruct((N_PAD, M), jnp.float32),
        ),
        mesh=mesh,
        compiler_params=pltpu.CompilerParams(use_tc_tiling_on_sc=False),
        scratch_types=[
            pltpu.VMEM((MAXCH, CHUNK), jnp.int32),
            pltpu.VMEM((MAXCH * CHUNK, M), jnp.float32),
            pltpu.VMEM((CHUNK, M), jnp.float32),
            pltpu.VMEM((ZROWS, M), jnp.float32),
            pltpu.VMEM_SHARED((N_PAD, M), jnp.float32),
            pltpu.VMEM_SHARED((N_PAD, M), jnp.float32),
            pltpu.SemaphoreType.DMA,
        ],
    )
    def scatter_kernel(msg_hbm, ei_hbm, seg0_hbm, seg1_hbm, cnt0_hbm,
                       cnt1_hbm, idx_v, msg_v, ones_v, zz_v, seg_sp, cnt_sp,
                       sem):
        cid = lax.axis_index("c")
        sid = lax.axis_index("s")
        wid = sid * NC + cid
        base = _worker_base(wid)
        col_hbm = ei_hbm.at[1]

        zrow = jnp.zeros((M,), jnp.float32)

        def zbody(i, c):
            zz_v[i, :] = zrow
            return c

        lax.fori_loop(0, ZROWS, zbody, 0)

        orow = jnp.ones((M,), jnp.float32)

        def obody(i, c):
            ones_v[i, :] = orow
            return c

        lax.fori_loop(0, CHUNK, obody, 0)

        rbase = sid * ZROWS
        pltpu.sync_copy(zz_v, seg_sp.at[pl.ds(rbase, ZROWS)])
        pltpu.sync_copy(zz_v, cnt_sp.at[pl.ds(rbase, ZROWS)])
        pltpu.sync_copy(col_hbm.at[pl.ds(base, TFULL)],
                        idx_v.at[pl.ds(0, TFULL)])
        pltpu.sync_copy(msg_hbm.at[pl.ds(base * CHUNK, TFULL * CHUNK)],
                        msg_v.at[pl.ds(0, TFULL * CHUNK)])

        @pl.when(wid < NEXTRA)
        def _():
            pltpu.sync_copy(col_hbm.at[base + TFULL], idx_v.at[TFULL])
            pltpu.sync_copy(
                msg_hbm.at[pl.ds((base + TFULL) * CHUNK, CHUNK)],
                msg_v.at[pl.ds(TFULL * CHUNK, CHUNK)])

        plsc.subcore_barrier()

        def sgroup(g, carry):
            t0 = g * FIRE
            descs = []
            for k in range(FIRE):
                t = t0 + k
                descs.append(pltpu.async_copy(
                    msg_v.at[pl.ds(t * CHUNK, CHUNK)],
                    seg_sp.at[idx_v.at[t]], sem, add=True))
                descs.append(pltpu.async_copy(
                    ones_v, cnt_sp.at[idx_v.at[t]], sem, add=True))
            for d in descs:
                d.wait()
            return carry

        lax.fori_loop(0, TFULL // FIRE, sgroup, 0)
        tail = []
        for t in range(FIRE * (TFULL // FIRE), TFULL):
            tail.append(pltpu.async_copy(
                msg_v.at[pl.ds(t * CHUNK, CHUNK)],
                seg_sp.at[idx_v.at[t]], sem, add=True))
            tail.append(pltpu.async_copy(
                ones_v, cnt_sp.at[idx_v.at[t]], sem, add=True))
        for d in tail:
            d.wait()

        @pl.when(wid < NEXTRA)
        def _():
            pltpu.async_copy(
                msg_v.at[pl.ds(TFULL * CHUNK, CHUNK)],
                seg_sp.at[idx_v.at[TFULL]], sem, add=True).wait()
            pltpu.async_copy(
                ones_v, cnt_sp.at[idx_v.at[TFULL]], sem, add=True).wait()

        plsc.subcore_barrier()

        pltpu.sync_copy(seg_sp.at[pl.ds(rbase, ZROWS)], zz_v)

        @pl.when(cid == 0)
        def _():
            pltpu.sync_copy(zz_v, seg0_hbm.at[pl.ds(rbase, ZROWS)])

        @pl.when(cid == 1)
        def _():
            pltpu.sync_copy(zz_v, seg1_hbm.at[pl.ds(rbase, ZROWS)])

        pltpu.sync_copy(cnt_sp.at[pl.ds(rbase, ZROWS)], zz_v)

        @pl.when(cid == 0)
        def _():
            pltpu.sync_copy(zz_v, cnt0_hbm.at[pl.ds(rbase, ZROWS)])

        @pl.when(cid == 1)
        def _():
            pltpu.sync_copy(zz_v, cnt1_hbm.at[pl.ds(rbase, ZROWS)])

    return scatter_kernel(msg, ei3)


def _ln_relu_packed(x, gt, bet, gsum, ubc):
    mu = jnp.dot(x, gsum, preferred_element_type=jnp.float32)
    m2 = jnp.dot(x * x, gsum, preferred_element_type=jnp.float32)
    var = m2 - mu * mu
    rs = lax.rsqrt(var + 1e-5)
    scale = jnp.dot(rs, ubc, preferred_element_type=jnp.float32)
    shift = jnp.dot(mu * rs, ubc, preferred_element_type=jnp.float32)
    return jnp.maximum((x * scale - shift) * gt + bet, 0.0)


def _tc_messages(ef8, rbas, src8, consts):
    def body(ef_ref, rbas_ref, src_ref, w1bd, r1bd, b1t, g1t, be1t,
             w2bd, b2t, g2t, be2t, w3bd, b3t, tbd, sbd, gsum, ubc, bmsg,
             out_ref):
        x = jnp.dot(ef_ref[...], w1bd[...], preferred_element_type=jnp.float32)
        x = x + jnp.dot(rbas_ref[...], r1bd[...],
                        preferred_element_type=jnp.float32) + b1t[...]
        x = _ln_relu_packed(x, g1t[...], be1t[...], gsum[...], ubc[...])
        x = jnp.dot(x, w2bd[...], preferred_element_type=jnp.float32) + b2t[...]
        x = _ln_relu_packed(x, g2t[...], be2t[...], gsum[...], ubc[...])
        y = jnp.dot(x, w3bd[...], preferred_element_type=jnp.float32) + b3t[...]
        srep = jnp.dot(src_ref[...], tbd[...],
                       preferred_element_type=jnp.float32)
        m = jnp.dot(y * srep, sbd[...], preferred_element_type=jnp.float32)
        out_ref[...] = m * jnp.dot(rbas_ref[...], bmsg[...],
                                   preferred_element_type=jnp.float32)

    eb128 = pl.BlockSpec((EBP, P * M), lambda i: (i, 0))
    eb16 = pl.BlockSpec((EBP, 2 * P), lambda i: (i, 0))

    def wspec(a):
        return pl.BlockSpec(a.shape, lambda i: (0,) * a.ndim)

    return pl.pallas_call(
        body,
        grid=(E8 // EBP,),
        in_specs=[eb128, eb16, eb128] + [wspec(w) for w in consts],
        out_specs=eb128,
        out_shape=jax.ShapeDtypeStruct((E8, P * M), jnp.float32),
    )(ef8, rbas, src8, *consts)


def _tc_finalize(seg0, seg1, cnt0, cnt1, h08, wst8):
    def body(s0_ref, s1_ref, c0_ref, c1_ref, h0_ref, w_ref, out_ref):
        sg = s0_ref[...] + s1_ref[...]
        ct = c0_ref[...] + c1_ref[...]
        sf = jnp.dot(h0_ref[...], w_ref[...], preferred_element_type=jnp.float32)
        out_ref[...] = sg / jnp.maximum(ct, 1.0) + jnp.where(ct > 0.0, sf, 0.0)

    node_spec = pl.BlockSpec((NP8, P * M), lambda: (0, 0))
    w_spec = pl.BlockSpec((P * M, P * M), lambda: (0, 0))
    return pl.pallas_call(
        body,
        in_specs=[node_spec, node_spec, node_spec, node_spec, node_spec,
                  w_spec],
        out_specs=node_spec,
        out_shape=jax.ShapeDtypeStruct((NP8, P * M), jnp.float32),
    )(seg0, seg1, cnt0, cnt1, h08, wst8)


def kernel(h0, r, basis_00, edge_index, edge_feat,
           W1, b1, g1, be1, W2, b2, g2, be2, W3, b3, W_self):
    f32 = jnp.float32
    h0f = h0.reshape(N_NODES, M)
    ei3 = edge_index.reshape(2, NCH, CHUNK)

    src = _sc_gather(ei3, h0f)

    eyeP = jnp.eye(P, dtype=f32)
    k = jnp.arange(OUT3)
    T = jnp.tile(jnp.eye(M, dtype=f32), (1, M))
    S = (k[:, None] // M == jnp.arange(M)[None, :]).astype(f32)
    r1bd = jnp.concatenate(
        [jnp.kron(eyeP, W1[M:]), jnp.zeros((P, P * MID), f32)], axis=0)
    bmsg = jnp.concatenate(
        [jnp.zeros((P, P * M), f32),
         jnp.kron(eyeP, jnp.ones((1, M), f32))], axis=0)
    consts = (
        jnp.kron(eyeP, W1[:M]),
        r1bd,
        jnp.tile(b1, P).reshape(1, P * MID),
        jnp.tile(g1, P).reshape(1, P * MID),
        jnp.tile(be1, P).reshape(1, P * MID),
        jnp.kron(eyeP, W2),
        jnp.tile(b2, P).reshape(1, P * MID),
        jnp.tile(g2, P).reshape(1, P * MID),
        jnp.tile(be2, P).reshape(1, P * MID),
        jnp.kron(eyeP, W3),
        jnp.tile(b3, P).reshape(1, P * OUT3),
        jnp.kron(eyeP, T),
        jnp.kron(eyeP, S),
        jnp.kron(eyeP, jnp.full((MID, 1), 1.0 / MID, f32)),
        jnp.kron(eyeP, jnp.ones((1, MID), f32)),
        bmsg,
    )
    rbas = jnp.concatenate(
        [r.reshape(E8, P), basis_00.reshape(E8, P)], axis=1)
    msg8 = _tc_messages(
        edge_feat.reshape(E8, P * M), rbas, src.reshape(E8, P * M), consts)

    seg0, seg1, cnt0, cnt1 = _sc_scatter(msg8.reshape(N_EDGES, M), ei3)

    h08 = jnp.concatenate(
        [h0f.reshape(N8, P * M), jnp.zeros((NP8 - N8, P * M), f32)])
    outp = _tc_finalize(
        seg0.reshape(NP8, P * M), seg1.reshape(NP8, P * M),
        cnt0.reshape(NP8, P * M), cnt1.reshape(NP8, P * M),
        h08, jnp.kron(eyeP, W_self[0].T))
    return outp[:N8].reshape(N_NODES, M, 1)


# speedup vs baseline: 5.9836x; 1.1005x over previous
import functools

import jax
import jax.numpy as jnp
from jax import lax
from jax.experimental import pallas as pl
from jax.experimental.pallas import tpu as pltpu
from jax.experimental.pallas import tpu_sc as plsc

N_NODES = 10000
N_EDGES = 160000
M = 16
MID = 32
OUT3 = 256

NC = 2
NS = 16
NW = NC * NS
CHUNK = 128
NCH = N_EDGES // CHUNK
TFULL = NCH // NW
NEXTRA = NCH - TFULL * NW
MAXCH = TFULL + 1
FIRE = 8
N_PAD = 10240
ZROWS = N_PAD // NS

P = 8
EB = 6400
EBP = EB // P
E8 = N_EDGES // P
N8 = N_NODES // P
NP8 = N_PAD * M // (P * M)


def _worker_base(wid):
    return wid * TFULL + jnp.minimum(wid, NEXTRA)


def _sc_gather(ei3, h0f):
    mesh = plsc.VectorSubcoreMesh(core_axis_name="c", subcore_axis_name="s")

    @functools.partial(
        pl.kernel,
        out_type=jax.ShapeDtypeStruct((N_EDGES, M), jnp.float32),
        mesh=mesh,
        compiler_params=pltpu.CompilerParams(use_tc_tiling_on_sc=False),
        scratch_types=[
            pltpu.VMEM((MAXCH, CHUNK), jnp.int32),
            pltpu.VMEM((MAXCH * CHUNK, M), jnp.float32),
            pltpu.SemaphoreType.DMA,
        ],
    )
    def gather_kernel(ei_hbm, h0_hbm, src_hbm, idx_v, rows_v, sem):
        cid = lax.axis_index("c")
        sid = lax.axis_index("s")
        wid = sid * NC + cid
        base = _worker_base(wid)
        row_hbm = ei_hbm.at[0]
        pltpu.sync_copy(row_hbm.at[pl.ds(base, TFULL)],
                        idx_v.at[pl.ds(0, TFULL)])

        def group(g, carry):
            t0 = g * FIRE
            descs = []
            for k in range(FIRE):
                t = t0 + k
                descs.append(pltpu.async_copy(
                    h0_hbm.at[idx_v.at[t]],
                    rows_v.at[pl.ds(t * CHUNK, CHUNK)],
                    sem))
            for d in descs:
                d.wait()
            return carry

        lax.fori_loop(0, TFULL // FIRE, group, 0)
        tail = []
        for t in range(FIRE * (TFULL // FIRE), TFULL):
            tail.append(pltpu.async_copy(
                h0_hbm.at[idx_v.at[t]],
                rows_v.at[pl.ds(t * CHUNK, CHUNK)],
                sem))
        for d in tail:
            d.wait()
        pltpu.sync_copy(rows_v.at[pl.ds(0, TFULL * CHUNK)],
                        src_hbm.at[pl.ds(base * CHUNK, TFULL * CHUNK)])

        @pl.when(wid < NEXTRA)
        def _():
            pltpu.sync_copy(row_hbm.at[base + TFULL], idx_v.at[TFULL])
            pltpu.async_copy(
                h0_hbm.at[idx_v.at[TFULL]],
                rows_v.at[pl.ds(TFULL * CHUNK, CHUNK)], sem).wait()
            pltpu.sync_copy(
                rows_v.at[pl.ds(TFULL * CHUNK, CHUNK)],
                src_hbm.at[pl.ds((base + TFULL) * CHUNK, CHUNK)])

    return gather_kernel(ei3, h0f)


def _sc_scatter(msg, ei3):
    mesh = plsc.VectorSubcoreMesh(core_axis_name="c", subcore_axis_name="s")

    @functools.partial(
        pl.kernel,
        out_type=(
            jax.ShapeDtypeStruct((N_PAD, M), jnp.float32),
            jax.ShapeDtypeStruct((N_PAD, M), jnp.float32),
            jax.ShapeDtypeStruct((N_PAD, M), jnp.float32),
            jax.ShapeDtypeStruct((N_PAD, M), jnp.float32),
        ),
        mesh=mesh,
        compiler_params=pltpu.CompilerParams(use_tc_tiling_on_sc=False),
        scratch_types=[
            pltpu.VMEM((MAXCH, CHUNK), jnp.int32),
            pltpu.VMEM((MAXCH * CHUNK, M), jnp.float32),
            pltpu.VMEM((CHUNK, M), jnp.float32),
            pltpu.VMEM((ZROWS, M), jnp.float32),
            pltpu.VMEM_SHARED((N_PAD, M), jnp.float32),
            pltpu.VMEM_SHARED((N_PAD, M), jnp.float32),
            pltpu.SemaphoreType.DMA,
        ],
    )
    def scatter_kernel(msg_hbm, ei_hbm, seg0_hbm, seg1_hbm, cnt0_hbm,
                       cnt1_hbm, idx_v, msg_v, ones_v, zz_v, seg_sp, cnt_sp,
                       sem):
        cid = lax.axis_index("c")
        sid = lax.axis_index("s")
        wid = sid * NC + cid
        base = _worker_base(wid)
        col_hbm = ei_hbm.at[1]

        zrow = jnp.zeros((M,), jnp.float32)

        def zbody(i, c):
            zz_v[i, :] = zrow
            return c

        lax.fori_loop(0, ZROWS, zbody, 0)

        orow = jnp.ones((M,), jnp.float32)

        def obody(i, c):
            ones_v[i, :] = orow
            return c

        lax.fori_loop(0, CHUNK, obody, 0)

        rbase = sid * ZROWS
        pltpu.sync_copy(zz_v, seg_sp.at[pl.ds(rbase, ZROWS)])
        pltpu.sync_copy(zz_v, cnt_sp.at[pl.ds(rbase, ZROWS)])
        pltpu.sync_copy(col_hbm.at[pl.ds(base, TFULL)],
                        idx_v.at[pl.ds(0, TFULL)])
        pltpu.sync_copy(msg_hbm.at[pl.ds(base * CHUNK, TFULL * CHUNK)],
                        msg_v.at[pl.ds(0, TFULL * CHUNK)])

        @pl.when(wid < NEXTRA)
        def _():
            pltpu.sync_copy(col_hbm.at[base + TFULL], idx_v.at[TFULL])
            pltpu.sync_copy(
                msg_hbm.at[pl.ds((base + TFULL) * CHUNK, CHUNK)],
                msg_v.at[pl.ds(TFULL * CHUNK, CHUNK)])

        plsc.subcore_barrier()

        def sgroup(g, carry):
            t0 = g * FIRE
            descs = []
            for k in range(FIRE):
                t = t0 + k
                descs.append(pltpu.async_copy(
                    msg_v.at[pl.ds(t * CHUNK, CHUNK)],
                    seg_sp.at[idx_v.at[t]], sem, add=True))
                descs.append(pltpu.async_copy(
                    ones_v, cnt_sp.at[idx_v.at[t]], sem, add=True))
            for d in descs:
                d.wait()
            return carry

        lax.fori_loop(0, TFULL // FIRE, sgroup, 0)
        tail = []
        for t in range(FIRE * (TFULL // FIRE), TFULL):
            tail.append(pltpu.async_copy(
                msg_v.at[pl.ds(t * CHUNK, CHUNK)],
                seg_sp.at[idx_v.at[t]], sem, add=True))
            tail.append(pltpu.async_copy(
                ones_v, cnt_sp.at[idx_v.at[t]], sem, add=True))
        for d in tail:
            d.wait()

        @pl.when(wid < NEXTRA)
        def _():
            pltpu.async_copy(
                msg_v.at[pl.ds(TFULL * CHUNK, CHUNK)],
                seg_sp.at[idx_v.at[TFULL]], sem, add=True).wait()
            pltpu.async_copy(
                ones_v, cnt_sp.at[idx_v.at[TFULL]], sem, add=True).wait()

        plsc.subcore_barrier()

        pltpu.sync_copy(seg_sp.at[pl.ds(rbase, ZROWS)], zz_v)

        @pl.when(cid == 0)
        def _():
            pltpu.sync_copy(zz_v, seg0_hbm.at[pl.ds(rbase, ZROWS)])

        @pl.when(cid == 1)
        def _():
            pltpu.sync_copy(zz_v, seg1_hbm.at[pl.ds(rbase, ZROWS)])

        pltpu.sync_copy(cnt_sp.at[pl.ds(rbase, ZROWS)], zz_v)

        @pl.when(cid == 0)
        def _():
            pltpu.sync_copy(zz_v, cnt0_hbm.at[pl.ds(rbase, ZROWS)])

        @pl.when(cid == 1)
        def _():
            pltpu.sync_copy(zz_v, cnt1_hbm.at[pl.ds(rbase, ZROWS)])

    return scatter_kernel(msg, ei3)


def _ln_relu_packed(x, gt, bet, gsum, ubc):
    mu = jnp.dot(x, gsum, preferred_element_type=jnp.float32)
    m2 = jnp.dot(x * x, gsum, preferred_element_type=jnp.float32)
    var = m2 - mu * mu
    rs = lax.rsqrt(var + 1e-5)
    scale = jnp.dot(rs, ubc, preferred_element_type=jnp.float32)
    shift = jnp.dot(mu * rs, ubc, preferred_element_type=jnp.float32)
    return jnp.maximum((x * scale - shift) * gt + bet, 0.0)


def _tc_messages(ef8, rbas, src8, consts):
    def body(ef_ref, rbas_ref, src_ref, w1bd, r1bd, b1t, g1t, be1t,
             w2bd, b2t, g2t, be2t, w3bd, b3t, tbd, s16c, gsum, ubc, bmsg,
             out_ref):
        x = jnp.dot(ef_ref[...], w1bd[...], preferred_element_type=jnp.float32)
        x = x + jnp.dot(rbas_ref[...], r1bd[...],
                        preferred_element_type=jnp.float32) + b1t[...]
        x = _ln_relu_packed(x, g1t[...], be1t[...], gsum[...], ubc[...])
        x = jnp.dot(x, w2bd[...], preferred_element_type=jnp.float32) + b2t[...]
        x = _ln_relu_packed(x, g2t[...], be2t[...], gsum[...], ubc[...])
        src = src_ref[...]
        s16 = s16c[...]
        parts = []
        for q in range(P):
            yq = jnp.dot(x, w3bd[:, q * OUT3:(q + 1) * OUT3],
                         preferred_element_type=jnp.float32)
            yq = yq + b3t[:, q * OUT3:(q + 1) * OUT3]
            sq = jnp.dot(src, tbd[:, q * OUT3:(q + 1) * OUT3],
                         preferred_element_type=jnp.float32)
            parts.append(jnp.dot(yq * sq, s16,
                                 preferred_element_type=jnp.float32))
        m = jnp.concatenate(parts, axis=1)
        out_ref[...] = m * jnp.dot(rbas_ref[...], bmsg[...],
                                   preferred_element_type=jnp.float32)

    eb128 = pl.BlockSpec((EBP, P * M), lambda i: (i, 0))
    eb16 = pl.BlockSpec((EBP, 2 * P), lambda i: (i, 0))

    def wspec(a):
        return pl.BlockSpec(a.shape, lambda i: (0,) * a.ndim)

    return pl.pallas_call(
        body,
        grid=(E8 // EBP,),
        in_specs=[eb128, eb16, eb128] + [wspec(w) for w in consts],
        out_specs=eb128,
        out_shape=jax.ShapeDtypeStruct((E8, P * M), jnp.float32),
    )(ef8, rbas, src8, *consts)


def _tc_finalize(seg0, seg1, cnt0, cnt1, h08, wst8):
    def body(s0_ref, s1_ref, c0_ref, c1_ref, h0_ref, w_ref, out_ref):
        sg = s0_ref[...] + s1_ref[...]
        ct = c0_ref[...] + c1_ref[...]
        sf = jnp.dot(h0_ref[...], w_ref[...], preferred_element_type=jnp.float32)
        out_ref[...] = sg / jnp.maximum(ct, 1.0) + jnp.where(ct > 0.0, sf, 0.0)

    node_spec = pl.BlockSpec((NP8, P * M), lambda: (0, 0))
    w_spec = pl.BlockSpec((P * M, P * M), lambda: (0, 0))
    return pl.pallas_call(
        body,
        in_specs=[node_spec, node_spec, node_spec, node_spec, node_spec,
                  w_spec],
        out_specs=node_spec,
        out_shape=jax.ShapeDtypeStruct((NP8, P * M), jnp.float32),
    )(seg0, seg1, cnt0, cnt1, h08, wst8)


def kernel(h0, r, basis_00, edge_index, edge_feat,
           W1, b1, g1, be1, W2, b2, g2, be2, W3, b3, W_self):
    f32 = jnp.float32
    h0f = h0.reshape(N_NODES, M)
    ei3 = edge_index.reshape(2, NCH, CHUNK)

    src = _sc_gather(ei3, h0f)

    eyeP = jnp.eye(P, dtype=f32)
    k = jnp.arange(OUT3)
    T = jnp.tile(jnp.eye(M, dtype=f32), (1, M))
    S = (k[:, None] // M == jnp.arange(M)[None, :]).astype(f32)
    r1bd = jnp.kron(jnp.eye(2 * P, P, dtype=f32), W1[M:])
    bmsg = jnp.kron(jnp.eye(2 * P, P, k=-P, dtype=f32),
                    jnp.ones((1, M), f32))
    consts = (
        jnp.kron(eyeP, W1[:M]),
        r1bd,
        jnp.tile(b1, P).reshape(1, P * MID),
        jnp.tile(g1, P).reshape(1, P * MID),
        jnp.tile(be1, P).reshape(1, P * MID),
        jnp.kron(eyeP, W2),
        jnp.tile(b2, P).reshape(1, P * MID),
        jnp.tile(g2, P).reshape(1, P * MID),
        jnp.tile(be2, P).reshape(1, P * MID),
        jnp.kron(eyeP, W3),
        jnp.tile(b3, P).reshape(1, P * OUT3),
        jnp.kron(eyeP, T),
        S,
        jnp.kron(eyeP, jnp.full((MID, 1), 1.0 / MID, f32)),
        jnp.kron(eyeP, jnp.ones((1, MID), f32)),
        bmsg,
    )
    rbas = jnp.concatenate(
        [r.reshape(E8, P), basis_00.reshape(E8, P)], axis=1)
    msg8 = _tc_messages(
        edge_feat.reshape(E8, P * M), rbas, src.reshape(E8, P * M), consts)

    seg0, seg1, cnt0, cnt1 = _sc_scatter(msg8.reshape(N_EDGES, M), ei3)

    h08 = jnp.concatenate(
        [h0f.reshape(N8, P * M), jnp.zeros((NP8 - N8, P * M), f32)])
    outp = _tc_finalize(
        seg0.reshape(NP8, P * M), seg1.reshape(NP8, P * M),
        cnt0.reshape(NP8, P * M), cnt1.reshape(NP8, P * M),
        h08, jnp.kron(eyeP, W_self[0].T))
    return outp[:N8].reshape(N_NODES, M, 1)


# speedup vs baseline: 6.1733x; 1.0317x over previous
import functools

import jax
import jax.numpy as jnp
from jax import lax
from jax.experimental import pallas as pl
from jax.experimental.pallas import tpu as pltpu
from jax.experimental.pallas import tpu_sc as plsc

N_NODES = 10000
N_EDGES = 160000
M = 16
MID = 32
OUT3 = 256

NC = 2
NS = 16
NW = NC * NS
CHUNK = 128
NCH = N_EDGES // CHUNK
TFULL = NCH // NW
NEXTRA = NCH - TFULL * NW
MAXCH = TFULL + 1
FIRE = 8
N_PAD = 10240
ZROWS = N_PAD // NS

P = 8
EB = 16000
EBP = EB // P
E8 = N_EDGES // P
N8 = N_NODES // P
NP8 = N_PAD * M // (P * M)


def _worker_base(wid):
    return wid * TFULL + jnp.minimum(wid, NEXTRA)


def _sc_gather(ei3, h0f):
    mesh = plsc.VectorSubcoreMesh(core_axis_name="c", subcore_axis_name="s")

    @functools.partial(
        pl.kernel,
        out_type=jax.ShapeDtypeStruct((N_EDGES, M), jnp.float32),
        mesh=mesh,
        compiler_params=pltpu.CompilerParams(use_tc_tiling_on_sc=False),
        scratch_types=[
            pltpu.VMEM((MAXCH, CHUNK), jnp.int32),
            pltpu.VMEM((MAXCH * CHUNK, M), jnp.float32),
            pltpu.SemaphoreType.DMA,
        ],
    )
    def gather_kernel(ei_hbm, h0_hbm, src_hbm, idx_v, rows_v, sem):
        cid = lax.axis_index("c")
        sid = lax.axis_index("s")
        wid = sid * NC + cid
        base = _worker_base(wid)
        row_hbm = ei_hbm.at[0]
        pltpu.sync_copy(row_hbm.at[pl.ds(base, TFULL)],
                        idx_v.at[pl.ds(0, TFULL)])

        def group(g, carry):
            descs = []
            for k in range(FIRE):
                t = g * FIRE + k
                descs.append(pltpu.async_copy(
                    h0_hbm.at[idx_v.at[t]],
                    rows_v.at[pl.ds(t * CHUNK, CHUNK)],
                    sem))
            for d in descs:
                d.wait()
            return carry

        lax.fori_loop(0, TFULL // FIRE, group, 0)
        tail = []
        for t in range(FIRE * (TFULL // FIRE), TFULL):
            tail.append(pltpu.async_copy(
                h0_hbm.at[idx_v.at[t]],
                rows_v.at[pl.ds(t * CHUNK, CHUNK)],
                sem))
        for d in tail:
            d.wait()
        pltpu.sync_copy(rows_v.at[pl.ds(0, TFULL * CHUNK)],
                        src_hbm.at[pl.ds(base * CHUNK, TFULL * CHUNK)])

        @pl.when(wid < NEXTRA)
        def _():
            pltpu.sync_copy(row_hbm.at[base + TFULL], idx_v.at[TFULL])
            pltpu.async_copy(
                h0_hbm.at[idx_v.at[TFULL]],
                rows_v.at[pl.ds(TFULL * CHUNK, CHUNK)], sem).wait()
            pltpu.sync_copy(
                rows_v.at[pl.ds(TFULL * CHUNK, CHUNK)],
                src_hbm.at[pl.ds((base + TFULL) * CHUNK, CHUNK)])

    return gather_kernel(ei3, h0f)


def _sc_scatter(msg, ei3):
    mesh = plsc.VectorSubcoreMesh(core_axis_name="c", subcore_axis_name="s")

    @functools.partial(
        pl.kernel,
        out_type=(
            jax.ShapeDtypeStruct((N_PAD, M), jnp.float32),
            jax.ShapeDtypeStruct((N_PAD, M), jnp.float32),
            jax.ShapeDtypeStruct((N_PAD, M), jnp.float32),
            jax.ShapeDtypeStruct((N_PAD, M), jnp.float32),
        ),
        mesh=mesh,
        compiler_params=pltpu.CompilerParams(use_tc_tiling_on_sc=False),
        scratch_types=[
            pltpu.VMEM((MAXCH, CHUNK), jnp.int32),
            pltpu.VMEM((MAXCH * CHUNK, M), jnp.float32),
            pltpu.VMEM((CHUNK, M), jnp.float32),
            pltpu.VMEM((ZROWS, M), jnp.float32),
            pltpu.VMEM_SHARED((N_PAD, M), jnp.float32),
            pltpu.VMEM_SHARED((N_PAD, M), jnp.float32),
            pltpu.SemaphoreType.DMA,
        ],
    )
    def scatter_kernel(msg_hbm, ei_hbm, seg0_hbm, seg1_hbm, cnt0_hbm,
                       cnt1_hbm, idx_v, msg_v, ones_v, zz_v, seg_sp, cnt_sp,
                       sem):
        cid = lax.axis_index("c")
        sid = lax.axis_index("s")
        wid = sid * NC + cid
        base = _worker_base(wid)
        col_hbm = ei_hbm.at[1]

        zrow = jnp.zeros((M,), jnp.float32)

        def zbody(i, c):
            zz_v[i, :] = zrow
            return c

        lax.fori_loop(0, ZROWS, zbody, 0)

        orow = jnp.ones((M,), jnp.float32)

        def obody(i, c):
            ones_v[i, :] = orow
            return c

        lax.fori_loop(0, CHUNK, obody, 0)

        rbase = sid * ZROWS
        pltpu.sync_copy(zz_v, seg_sp.at[pl.ds(rbase, ZROWS)])
        pltpu.sync_copy(zz_v, cnt_sp.at[pl.ds(rbase, ZROWS)])
        pltpu.sync_copy(col_hbm.at[pl.ds(base, TFULL)],
                        idx_v.at[pl.ds(0, TFULL)])
        pltpu.sync_copy(msg_hbm.at[pl.ds(base * CHUNK, TFULL * CHUNK)],
                        msg_v.at[pl.ds(0, TFULL * CHUNK)])

        @pl.when(wid < NEXTRA)
        def _():
            pltpu.sync_copy(col_hbm.at[base + TFULL], idx_v.at[TFULL])
            pltpu.sync_copy(
                msg_hbm.at[pl.ds((base + TFULL) * CHUNK, CHUNK)],
                msg_v.at[pl.ds(TFULL * CHUNK, CHUNK)])

        plsc.subcore_barrier()

        def sgroup(g, carry):
            t0 = g * FIRE
            descs = []
            for k in range(FIRE):
                t = t0 + k
                descs.append(pltpu.async_copy(
                    msg_v.at[pl.ds(t * CHUNK, CHUNK)],
                    seg_sp.at[idx_v.at[t]], sem, add=True))
                descs.append(pltpu.async_copy(
                    ones_v, cnt_sp.at[idx_v.at[t]], sem, add=True))
            for d in descs:
                d.wait()
            return carry

        lax.fori_loop(0, TFULL // FIRE, sgroup, 0)
        tail = []
        for t in range(FIRE * (TFULL // FIRE), TFULL):
            tail.append(pltpu.async_copy(
                msg_v.at[pl.ds(t * CHUNK, CHUNK)],
                seg_sp.at[idx_v.at[t]], sem, add=True))
            tail.append(pltpu.async_copy(
                ones_v, cnt_sp.at[idx_v.at[t]], sem, add=True))
        for d in tail:
            d.wait()

        @pl.when(wid < NEXTRA)
        def _():
            pltpu.async_copy(
                msg_v.at[pl.ds(TFULL * CHUNK, CHUNK)],
                seg_sp.at[idx_v.at[TFULL]], sem, add=True).wait()
            pltpu.async_copy(
                ones_v, cnt_sp.at[idx_v.at[TFULL]], sem, add=True).wait()

        plsc.subcore_barrier()

        pltpu.sync_copy(seg_sp.at[pl.ds(rbase, ZROWS)], zz_v)

        @pl.when(cid == 0)
        def _():
            pltpu.sync_copy(zz_v, seg0_hbm.at[pl.ds(rbase, ZROWS)])

        @pl.when(cid == 1)
        def _():
            pltpu.sync_copy(zz_v, seg1_hbm.at[pl.ds(rbase, ZROWS)])

        pltpu.sync_copy(cnt_sp.at[pl.ds(rbase, ZROWS)], zz_v)

        @pl.when(cid == 0)
        def _():
            pltpu.sync_copy(zz_v, cnt0_hbm.at[pl.ds(rbase, ZROWS)])

        @pl.when(cid == 1)
        def _():
            pltpu.sync_copy(zz_v, cnt1_hbm.at[pl.ds(rbase, ZROWS)])

    return scatter_kernel(msg, ei3)


def _ln_relu_packed(x, gt, bet, gsum, ubc):
    mu = jnp.dot(x, gsum, preferred_element_type=jnp.float32)
    m2 = jnp.dot(x * x, gsum, preferred_element_type=jnp.float32)
    var = m2 - mu * mu
    rs = lax.rsqrt(var + 1e-5)
    scale = jnp.dot(rs, ubc, preferred_element_type=jnp.float32)
    shift = jnp.dot(mu * rs, ubc, preferred_element_type=jnp.float32)
    return jnp.maximum((x * scale - shift) * gt + bet, 0.0)


def _tc_messages(ef8, rbas, src8, consts):
    def body(ef_ref, rbas_ref, src_ref, w1bd, r1bd, b1t, g1t, be1t,
             w2bd, b2t, g2t, be2t, w3bd, b3t, tbd, s16c, gsum, ubc, bmsg,
             out_ref):
        x = jnp.dot(ef_ref[...], w1bd[...], preferred_element_type=jnp.float32)
        x = x + jnp.dot(rbas_ref[...], r1bd[...],
                        preferred_element_type=jnp.float32) + b1t[...]
        x = _ln_relu_packed(x, g1t[...], be1t[...], gsum[...], ubc[...])
        x = jnp.dot(x, w2bd[...], preferred_element_type=jnp.float32) + b2t[...]
        x = _ln_relu_packed(x, g2t[...], be2t[...], gsum[...], ubc[...])
        src = src_ref[...]
        s16 = s16c[...]
        parts = []
        for q in range(P):
            yq = jnp.dot(x, w3bd[:, q * OUT3:(q + 1) * OUT3],
                         preferred_element_type=jnp.float32)
            yq = yq + b3t[:, q * OUT3:(q + 1) * OUT3]
            sq = jnp.dot(src, tbd[:, q * OUT3:(q + 1) * OUT3],
                         preferred_element_type=jnp.float32)
            parts.append(jnp.dot(yq * sq, s16,
                                 preferred_element_type=jnp.float32))
        m = jnp.concatenate(parts, axis=1)
        out_ref[...] = m * jnp.dot(rbas_ref[...], bmsg[...],
                                   preferred_element_type=jnp.float32)

    eb128 = pl.BlockSpec((EBP, P * M), lambda i: (i, 0))
    eb16 = pl.BlockSpec((EBP, 2 * P), lambda i: (i, 0))

    def wspec(a):
        return pl.BlockSpec(a.shape, lambda i: (0,) * a.ndim)

    return pl.pallas_call(
        body,
        grid=(E8 // EBP,),
        in_specs=[eb128, eb16, eb128] + [wspec(w) for w in consts],
        out_specs=eb128,
        out_shape=jax.ShapeDtypeStruct((E8, P * M), jnp.float32),
    )(ef8, rbas, src8, *consts)


def _tc_finalize(seg0, seg1, cnt0, cnt1, h08, wst8):
    def body(s0_ref, s1_ref, c0_ref, c1_ref, h0_ref, w_ref, out_ref):
        sg = s0_ref[...] + s1_ref[...]
        ct = c0_ref[...] + c1_ref[...]
        sf = jnp.dot(h0_ref[...], w_ref[...], preferred_element_type=jnp.float32)
        out_ref[...] = sg / jnp.maximum(ct, 1.0) + jnp.where(ct > 0.0, sf, 0.0)

    node_spec = pl.BlockSpec((NP8, P * M), lambda: (0, 0))
    w_spec = pl.BlockSpec((P * M, P * M), lambda: (0, 0))
    return pl.pallas_call(
        body,
        in_specs=[node_spec, node_spec, node_spec, node_spec, node_spec,
                  w_spec],
        out_specs=node_spec,
        out_shape=jax.ShapeDtypeStruct((NP8, P * M), jnp.float32),
    )(seg0, seg1, cnt0, cnt1, h08, wst8)


def kernel(h0, r, basis_00, edge_index, edge_feat,
           W1, b1, g1, be1, W2, b2, g2, be2, W3, b3, W_self):
    f32 = jnp.float32
    h0f = h0.reshape(N_NODES, M)
    ei3 = edge_index.reshape(2, NCH, CHUNK)

    src = _sc_gather(ei3, h0f)

    eyeP = jnp.eye(P, dtype=f32)
    k = jnp.arange(OUT3)
    T = jnp.tile(jnp.eye(M, dtype=f32), (1, M))
    S = (k[:, None] // M == jnp.arange(M)[None, :]).astype(f32)
    r1bd = jnp.kron(jnp.eye(2 * P, P, dtype=f32), W1[M:])
    bmsg = jnp.kron(jnp.eye(2 * P, P, k=-P, dtype=f32),
                    jnp.ones((1, M), f32))
    consts = (
        jnp.kron(eyeP, W1[:M]),
        r1bd,
        jnp.tile(b1, P).reshape(1, P * MID),
        jnp.tile(g1, P).reshape(1, P * MID),
        jnp.tile(be1, P).reshape(1, P * MID),
        jnp.kron(eyeP, W2),
        jnp.tile(b2, P).reshape(1, P * MID),
        jnp.tile(g2, P).reshape(1, P * MID),
        jnp.tile(be2, P).reshape(1, P * MID),
        jnp.kron(eyeP, W3),
        jnp.tile(b3, P).reshape(1, P * OUT3),
        jnp.kron(eyeP, T),
        S,
        jnp.kron(eyeP, jnp.full((MID, 1), 1.0 / MID, f32)),
        jnp.kron(eyeP, jnp.ones((1, MID), f32)),
        bmsg,
    )
    rbas = jnp.concatenate(
        [r.reshape(E8, P), basis_00.reshape(E8, P)], axis=1)
    msg8 = _tc_messages(
        edge_feat.reshape(E8, P * M), rbas, src.reshape(E8, P * M), consts)

    seg0, seg1, cnt0, cnt1 = _sc_scatter(msg8.reshape(N_EDGES, M), ei3)

    h08 = jnp.concatenate(
        [h0f.reshape(N8, P * M), jnp.zeros((NP8 - N8, P * M), f32)])
    outp = _tc_finalize(
        seg0.reshape(NP8, P * M), seg1.reshape(NP8, P * M),
        cnt0.reshape(NP8, P * M), cnt1.reshape(NP8, P * M),
        h08, jnp.kron(eyeP, W_self[0].T))
    return outp[:N8].reshape(N_NODES, M, 1)
